# baseline (speedup 1.0000x reference)
import functools
import jax
import jax.numpy as jnp
from jax import lax
from jax.experimental import pallas as pl
from jax.experimental.pallas import tpu as pltpu
from jax.experimental.pallas import tpu_sc as plsc

NBINS = 100
L = 16
ROWS_PER_W = 4
HALF = 128


def _table_kernel(t_ref, mean_ref, std_ref, scale_ref, shift_ref):
    for r in range(128):
        tb = (t_ref[r] * NBINS).astype(jnp.int32)
        tb = jnp.where(tb == NBINS, NBINS - 1, tb)
        m = mean_ref[tb]
        s = std_ref[tb]
        scale_ref[r] = jnp.full((128,), 1.0 / s, dtype=jnp.float32)
        shift_ref[r] = jnp.full((128,), m, dtype=jnp.float32)


def _make_tables(t, data_mean, data_std):
    grid_spec = pltpu.PrefetchScalarGridSpec(
        num_scalar_prefetch=3,
        grid=(1,),
        in_specs=[],
        out_specs=[pl.BlockSpec((128, 128), lambda *_: (0, 0)),
                   pl.BlockSpec((128, 128), lambda *_: (0, 0))],
    )
    return pl.pallas_call(
        _table_kernel,
        grid_spec=grid_spec,
        out_shape=[jax.ShapeDtypeStruct((128, 128), jnp.float32),
                   jax.ShapeDtypeStruct((128, 128), jnp.float32)],
    )(t, data_mean, data_std)


def _sc_body(x_hbm, scale_hbm, shift_hbm, out_hbm,
             scale_v, shift_v, in_a):
    c = lax.axis_index("c")
    s = lax.axis_index("s")
    wid = s * 2 + c

    def row_loop(j, _):
        r = wid * ROWS_PER_W + j
        pltpu.sync_copy(scale_hbm.at[r], scale_v)
        pltpu.sync_copy(shift_hbm.at[r], shift_v)

        def chunk_loop(k, _):
            ch = k // 2
            h = (k % 2) * HALF
            pltpu.sync_copy(x_hbm.at[r, ch, pl.ds(h, HALF)], in_a)

            def col_loop(cg, _):
                col = cg * L
                rr = scale_v[pl.ds(0, L)]
                mm = shift_v[pl.ds(0, L)]
                for row in range(HALF):
                    v = in_a[row, pl.ds(col, L)]
                    in_a[row, pl.ds(col, L)] = (v - mm) * rr
                return 0
            lax.fori_loop(0, 256 // L, col_loop, 0)
            pltpu.sync_copy(in_a, out_hbm.at[r, ch, pl.ds(h, HALF)])
            return 0
        lax.fori_loop(0, 8, chunk_loop, 0)
        return 0
    lax.fori_loop(0, ROWS_PER_W, row_loop, 0)


def kernel(x_t, t, data_mean, data_std):
    scale, shift = _make_tables(t, data_mean, data_std)
    mesh = plsc.VectorSubcoreMesh(core_axis_name="c", subcore_axis_name="s")
    run = functools.partial(
        pl.kernel,
        mesh=mesh,
        out_type=jax.ShapeDtypeStruct(x_t.shape, jnp.float32),
        scratch_types=[
            pltpu.VMEM((128,), jnp.float32),
            pltpu.VMEM((128,), jnp.float32),
            pltpu.VMEM((HALF, 256), jnp.float32),
        ],
    )(_sc_body)
    return run(x_t, scale, shift)
